# Initial kernel scaffold; baseline (speedup 1.0000x reference)
import jax
import jax.numpy as jnp
from jax.experimental import pallas as pl


def kernel(x, edge_index, W1, a_src1, a_dst1, b1, W2, a_src2, a_dst2, b2, W3, a_src3, a_dst3, b3, Wc, bc):
    raise NotImplementedError("write your pallas kernel here")


# speedup vs baseline: 1.2904x; 1.2904x over previous
import jax
import jax.numpy as jnp
from jax.experimental import pallas as pl


def _gat(x, src, dst, W, a_src, a_dst, b):
    N = x.shape[0]
    h = x @ W.T
    a_s = h @ a_src
    a_d = h @ a_dst
    A = jnp.max(a_s)
    c = jnp.maximum(A + a_d, 0.2 * (A + a_d))
    s = a_s[src] + a_d[dst]
    e = jnp.maximum(s, 0.2 * s)
    p = jnp.exp(e - c[dst])
    ss = a_s + a_d
    es = jnp.maximum(ss, 0.2 * ss)
    ps = jnp.exp(es - c)
    denom = jax.ops.segment_sum(p, dst, num_segments=N) + ps
    num = jax.ops.segment_sum(h[src] * p[:, None], dst, num_segments=N) + ps[:, None] * h
    return num / denom[:, None] + b


def kernel(x, edge_index, W1, a_src1, a_dst1, b1, W2, a_src2, a_dst2, b2,
           W3, a_src3, a_dst3, b3, Wc, bc):
    src, dst = edge_index[0], edge_index[1]
    h = jax.nn.relu(_gat(x, src, dst, W1, a_src1, a_dst1, b1))
    h = jax.nn.relu(_gat(h, src, dst, W2, a_src2, a_dst2, b2))
    h = jax.nn.relu(_gat(h, src, dst, W3, a_src3, a_dst3, b3))
    return (h @ Wc.T + bc, h)


# speedup vs baseline: 34.4919x; 26.7303x over previous
import functools

import jax
import jax.numpy as jnp
from jax import lax
from jax.experimental import pallas as pl
from jax.experimental.pallas import tpu as pltpu
from jax.experimental.pallas import tpu_sc as plsc

N = 10000
E = 320000
D_IN = 128
NP = 10240
NC, NS, L = 2, 16, 16
NW = NC * NS
CK = 128
CHUNKS = 80
E_TILE = CHUNKS * CK
E_PAD = E_TILE * NW
ROWS2D = E_PAD // CK
RB = NP // 4
SLICE = NP // NS
DUMMY = N

f32 = jnp.float32
i32 = jnp.int32


def _dense_first_body(x_ref, w_ref, asv_ref, adv_ref, h_ref, al_ref):
    g = jnp.dot(x_ref[...], w_ref[...].T, preferred_element_type=f32)
    h_ref[...] = g
    a0 = jnp.dot(g, asv_ref[...], preferred_element_type=f32)
    a1 = jnp.dot(g, adv_ref[...], preferred_element_type=f32)
    al_ref[...] = jnp.concatenate([a0, a1], axis=1)


def _dense_first(xp, W, a_src, a_dst, H):
    return pl.pallas_call(
        _dense_first_body,
        grid=(NP // RB,),
        in_specs=[
            pl.BlockSpec((RB, D_IN), lambda i: (i, 0)),
            pl.BlockSpec((H, D_IN), lambda i: (0, 0)),
            pl.BlockSpec((H, 1), lambda i: (0, 0)),
            pl.BlockSpec((H, 1), lambda i: (0, 0)),
        ],
        out_specs=[
            pl.BlockSpec((RB, H), lambda i: (i, 0)),
            pl.BlockSpec((RB, 2), lambda i: (i, 0)),
        ],
        out_shape=[
            jax.ShapeDtypeStruct((NP, H), f32),
            jax.ShapeDtypeStruct((NP, 2), f32),
        ],
    )(xp, W, a_src.reshape(H, 1), a_dst.reshape(H, 1))


def _stab_body(al_ref, c_ref, ps_ref):
    a_s = al_ref[:, 0:1]
    a_d = al_ref[:, 1:2]
    A = jnp.max(a_s)
    t = A + a_d
    c = jnp.maximum(t, 0.2 * t)
    ss = a_s + a_d
    es = jnp.maximum(ss, 0.2 * ss)
    c_ref[...] = c
    ps_ref[...] = jnp.exp(es - c)


def _stab(al):
    return pl.pallas_call(
        _stab_body,
        out_shape=[
            jax.ShapeDtypeStruct((NP, 1), f32),
            jax.ShapeDtypeStruct((NP, 1), f32),
        ],
    )(al)


def _combine_body(o_ref, d_ref, ps_ref, h_ref, b_ref, w_ref, asv_ref,
                  adv_ref, f_ref, hn_ref, al_ref):
    ps = ps_ref[...]
    num = o_ref[0] + o_ref[1] + ps * h_ref[...]
    den = d_ref[0] + d_ref[1] + ps
    f = jnp.maximum(num / den + b_ref[...], 0.0)
    f_ref[...] = f
    g = jnp.dot(f, w_ref[...].T, preferred_element_type=f32)
    hn_ref[...] = g
    a0 = jnp.dot(g, asv_ref[...], preferred_element_type=f32)
    a1 = jnp.dot(g, adv_ref[...], preferred_element_type=f32)
    al_ref[...] = jnp.concatenate([a0, a1], axis=1)


def _combine(o_parts, d_parts, ps, h_prev, b, W, a_src, a_dst, Hp, Hn):
    return pl.pallas_call(
        _combine_body,
        grid=(NP // RB,),
        in_specs=[
            pl.BlockSpec((2, RB, Hp), lambda i: (0, i, 0)),
            pl.BlockSpec((2, RB, 1), lambda i: (0, i, 0)),
            pl.BlockSpec((RB, 1), lambda i: (i, 0)),
            pl.BlockSpec((RB, Hp), lambda i: (i, 0)),
            pl.BlockSpec((1, Hp), lambda i: (0, 0)),
            pl.BlockSpec((Hn, Hp), lambda i: (0, 0)),
            pl.BlockSpec((Hn, 1), lambda i: (0, 0)),
            pl.BlockSpec((Hn, 1), lambda i: (0, 0)),
        ],
        out_specs=[
            pl.BlockSpec((RB, Hp), lambda i: (i, 0)),
            pl.BlockSpec((RB, Hn), lambda i: (i, 0)),
            pl.BlockSpec((RB, 2), lambda i: (i, 0)),
        ],
        out_shape=[
            jax.ShapeDtypeStruct((NP, Hp), f32),
            jax.ShapeDtypeStruct((NP, Hn), f32),
            jax.ShapeDtypeStruct((NP, 2), f32),
        ],
    )(o_parts, d_parts.reshape(2, NP, 1), ps, h_prev, b.reshape(1, Hp),
      W, a_src.reshape(Hn, 1), a_dst.reshape(Hn, 1))


def _final_body(o_ref, d_ref, ps_ref, h_ref, b_ref, wc_ref, bc_ref,
                hf_ref, out_ref):
    ps = ps_ref[...]
    num = o_ref[0] + o_ref[1] + ps * h_ref[...]
    den = d_ref[0] + d_ref[1] + ps
    hf = jnp.maximum(num / den + b_ref[...], 0.0)
    hf_ref[...] = hf
    out_ref[...] = (
        jnp.dot(hf, wc_ref[...].T, preferred_element_type=f32) + bc_ref[...])


def _final(o_parts, d_parts, ps, h_prev, b, Wc, bc, Hp, Hc):
    return pl.pallas_call(
        _final_body,
        grid=(NP // RB,),
        in_specs=[
            pl.BlockSpec((2, RB, Hp), lambda i: (0, i, 0)),
            pl.BlockSpec((2, RB, 1), lambda i: (0, i, 0)),
            pl.BlockSpec((RB, 1), lambda i: (i, 0)),
            pl.BlockSpec((RB, Hp), lambda i: (i, 0)),
            pl.BlockSpec((1, Hp), lambda i: (0, 0)),
            pl.BlockSpec((Hc, Hp), lambda i: (0, 0)),
            pl.BlockSpec((1, Hc), lambda i: (0, 0)),
        ],
        out_specs=[
            pl.BlockSpec((RB, Hp), lambda i: (i, 0)),
            pl.BlockSpec((RB, Hc), lambda i: (i, 0)),
        ],
        out_shape=[
            jax.ShapeDtypeStruct((NP, Hp), f32),
            jax.ShapeDtypeStruct((NP, Hc), f32),
        ],
    )(o_parts, d_parts.reshape(2, NP, 1), ps, h_prev, b.reshape(1, Hp),
      Wc, bc.reshape(1, Hc))


def _make_sc_edge(H):
    mesh = plsc.VectorSubcoreMesh(core_axis_name="c", subcore_axis_name="s")

    def body(src_hbm, dst_hbm, as_hbm, ad_hbm, c_hbm, h_hbm, z2_hbm, z1_hbm,
             o_hbm, d_hbm,
             out_sp, den_sp, src_t, dst_t, as_t, ad_t, c_t, p_t, rows_t):
        cid = lax.axis_index("c")
        sid = lax.axis_index("s")
        w = sid * NC + cid
        base = sid * SLICE
        pltpu.sync_copy(z2_hbm.at[pl.ds(base, SLICE)],
                        out_sp.at[pl.ds(base, SLICE)])
        pltpu.sync_copy(z1_hbm.at[pl.ds(base, SLICE)],
                        den_sp.at[pl.ds(base, SLICE)])
        pltpu.sync_copy(as_hbm, as_t)
        pltpu.sync_copy(ad_hbm, ad_t)
        pltpu.sync_copy(c_hbm, c_t)
        row0 = w * CHUNKS
        pltpu.sync_copy(src_hbm.at[pl.ds(row0, CHUNKS)], src_t)
        pltpu.sync_copy(dst_hbm.at[pl.ds(row0, CHUNKS)], dst_t)
        plsc.subcore_barrier()

        def chunk(j, carry):
            for i in range(CK // L):
                is_v = src_t[j, pl.ds(i * L, L)]
                id_v = dst_t[j, pl.ds(i * L, L)]
                as_v = plsc.load_gather(as_t, [is_v])
                ad_v = plsc.load_gather(ad_t, [id_v])
                c_v = plsc.load_gather(c_t, [id_v])
                s = as_v + ad_v
                e = jnp.maximum(s, 0.2 * s)
                p_t[pl.ds(i * L, L)] = jnp.exp(e - c_v)
            pltpu.sync_copy(h_hbm.at[src_t.at[j]], rows_t)

            def scale(k, c2):
                pk = plsc.load_gather(p_t, [jnp.full((L,), 0, i32) + k])
                for u in range(H // L):
                    rows_t[k, pl.ds(u * L, L)] = (
                        rows_t[k, pl.ds(u * L, L)] * pk)
                return c2

            lax.fori_loop(0, CK, scale, 0)
            pltpu.sync_copy(rows_t, out_sp.at[dst_t.at[j]], add=True)
            pltpu.sync_copy(p_t, den_sp.at[dst_t.at[j]], add=True)
            return carry

        lax.fori_loop(0, CHUNKS, chunk, 0)
        plsc.subcore_barrier()
        pltpu.sync_copy(out_sp.at[pl.ds(base, SLICE)],
                        o_hbm.at[cid, pl.ds(base, SLICE)])
        pltpu.sync_copy(den_sp.at[pl.ds(base, SLICE)],
                        d_hbm.at[cid, pl.ds(base, SLICE)])

    return pl.kernel(
        body,
        out_type=[
            jax.ShapeDtypeStruct((NC, NP, H), f32),
            jax.ShapeDtypeStruct((NC, NP), f32),
        ],
        mesh=mesh,
        compiler_params=pltpu.CompilerParams(
            needs_layout_passes=False, use_tc_tiling_on_sc=False),
        scratch_types=[
            pltpu.VMEM_SHARED((NP, H), f32),
            pltpu.VMEM_SHARED((NP,), f32),
            pltpu.VMEM((CHUNKS, CK), i32),
            pltpu.VMEM((CHUNKS, CK), i32),
            pltpu.VMEM((NP,), f32),
            pltpu.VMEM((NP,), f32),
            pltpu.VMEM((NP,), f32),
            pltpu.VMEM((CK,), f32),
            pltpu.VMEM((CK, H), f32),
        ],
    )


_sc_edge_16 = _make_sc_edge(16)
_sc_edge_32 = _make_sc_edge(32)


def _sc_edge(H, srcp, dstp, al, c, h, z2, z1):
    fn = _sc_edge_16 if H == 16 else _sc_edge_32
    as_f = al[:, 0].reshape(NP)
    ad_f = al[:, 1].reshape(NP)
    return fn(srcp, dstp, as_f, ad_f, c.reshape(NP), h, z2[:, :H], z1)


def kernel(x, edge_index, W1, a_src1, a_dst1, b1, W2, a_src2, a_dst2, b2,
           W3, a_src3, a_dst3, b3, Wc, bc):
    src = edge_index[0]
    dst = edge_index[1]
    pad = E_PAD - E
    srcp = jnp.concatenate([src, jnp.full((pad,), DUMMY, i32)]).reshape(
        ROWS2D, CK)
    dstp = jnp.concatenate([dst, jnp.full((pad,), DUMMY, i32)]).reshape(
        ROWS2D, CK)
    xp = jnp.pad(x, ((0, NP - N), (0, 0)))
    z2 = jnp.zeros((NP, 32), f32)
    z1 = jnp.zeros((NP,), f32)

    h1, al1 = _dense_first(xp, W1, a_src1, a_dst1, 16)
    c1, ps1 = _stab(al1)
    o1, d1 = _sc_edge(16, srcp, dstp, al1, c1, h1, z2, z1)

    _, h2, al2 = _combine(o1, d1, ps1, h1, b1, W2, a_src2, a_dst2, 16, 32)
    c2, ps2 = _stab(al2)
    o2, d2 = _sc_edge(32, srcp, dstp, al2, c2, h2, z2, z1)

    _, h3, al3 = _combine(o2, d2, ps2, h2, b2, W3, a_src3, a_dst3, 32, 32)
    c3, ps3 = _stab(al3)
    o3, d3 = _sc_edge(32, srcp, dstp, al3, c3, h3, z2, z1)

    hf, logits = _final(o3, d3, ps3, h3, b3, Wc, bc, 32, 32)
    return (logits[:N], hf[:N])
